# baseline (speedup 1.0000x reference)
import functools

import jax
import jax.numpy as jnp
from jax import lax
from jax.experimental import pallas as pl
from jax.experimental.pallas import tpu as pltpu
from jax.experimental.pallas import tpu_sc as plsc

B, C, U, D, S = 1024, 256, 128, 16, 16
NC, NS = 2, 16
L = 16
NBLK = U // L


def _sc_scatter_body(dend_hbm, dw_hbm, wt_hbm, dend_v, dw_v, w_v):
    wid = lax.axis_index("s") * NC + lax.axis_index("c")

    @pl.when(wid < NBLK)
    def _():
        u0 = wid * L
        pltpu.sync_copy(dend_hbm, dend_v)
        pltpu.sync_copy(dw_hbm, dw_v)
        rows = lax.iota(jnp.int32, L)
        for s in range(S):
            val = dw_v[s, pl.ds(u0, L)]
            for d in range(D):
                idx = dend_v[d * S + s, pl.ds(u0, L)]
                plsc.store_scatter(w_v, [rows, idx], val)
        pltpu.sync_copy(w_v, wt_hbm.at[pl.ds(u0, L)])


@functools.partial(
    pl.kernel,
    mesh=plsc.VectorSubcoreMesh(core_axis_name="c", subcore_axis_name="s"),
    out_type=jax.ShapeDtypeStruct((U, C), jnp.float32),
    scratch_types=[
        pltpu.VMEM((D * S, U), jnp.int32),
        pltpu.VMEM((S, U), jnp.float32),
        pltpu.VMEM((L, C), jnp.float32),
    ],
    compiler_params=pltpu.CompilerParams(needs_layout_passes=False),
)
def _sc_scatter(dend_hbm, dw_hbm, wt_hbm, dend_v, dw_v, w_v):
    _sc_scatter_body(dend_hbm, dw_hbm, wt_hbm, dend_v, dw_v, w_v)


def _tc_matmul_body(x_ref, wt_ref, kw_ref, b_ref, o_ref):
    acc = lax.dot_general(
        x_ref[:], wt_ref[:], (((1,), (1,)), ((), ())),
        preferred_element_type=jnp.float32)
    o_ref[:] = acc * kw_ref[:] + b_ref[:]


def _tc_matmul(x, wt, kw, b2):
    return pl.pallas_call(
        _tc_matmul_body,
        out_shape=jax.ShapeDtypeStruct((B, U), jnp.float32),
    )(x, wt, kw, b2)


def kernel(x, dendrites, kernel, dendriticW, bias):
    wt = _sc_scatter(dendrites.reshape(D * S, U), dendriticW)
    return _tc_matmul(x, wt, kernel, bias.reshape(1, U))


# speedup vs baseline: 1.2324x; 1.2324x over previous
import functools

import jax
import jax.numpy as jnp
from jax import lax
from jax.experimental import pallas as pl
from jax.experimental.pallas import tpu as pltpu
from jax.experimental.pallas import tpu_sc as plsc

B, C, U, D, S = 1024, 256, 128, 16, 16
NC, NS = 2, 16
NW = NC * NS
UPW = U // NW
L = 16


def _sc_scatter_body(duds_hbm, dw_hbm, wt_hbm, idx_v, dw_v, w_v, sem1, sem2):
    wid = lax.axis_index("s") * NC + lax.axis_index("c")
    base = wid * UPW
    cp1 = pltpu.async_copy(
        duds_hbm.at[pl.ds(base * D * S, UPW * D * S)], idx_v, sem1)
    cp2 = pltpu.async_copy(dw_hbm, dw_v, sem2)
    cp1.wait()
    cp2.wait()
    lane = lax.iota(jnp.int32, L)
    for j in range(UPW):
        rows = lane * 0 + j
        w16 = plsc.load_gather(dw_v, [lane, lane * 0 + (base + j)])
        for d in range(D):
            idx = idx_v[pl.ds(j * D * S + d * S, L)]
            plsc.store_scatter(w_v, [rows, idx], w16)
    pltpu.sync_copy(w_v, wt_hbm.at[pl.ds(base, UPW)])


@functools.partial(
    pl.kernel,
    mesh=plsc.VectorSubcoreMesh(core_axis_name="c", subcore_axis_name="s"),
    out_type=jax.ShapeDtypeStruct((U, C), jnp.float32),
    scratch_types=[
        pltpu.VMEM((UPW * D * S,), jnp.int32),
        pltpu.VMEM((S, U), jnp.float32),
        pltpu.VMEM((UPW, C), jnp.float32),
        pltpu.SemaphoreType.DMA,
        pltpu.SemaphoreType.DMA,
    ],
    compiler_params=pltpu.CompilerParams(needs_layout_passes=False),
)
def _sc_scatter(duds_hbm, dw_hbm, wt_hbm, idx_v, dw_v, w_v, sem1, sem2):
    _sc_scatter_body(duds_hbm, dw_hbm, wt_hbm, idx_v, dw_v, w_v, sem1, sem2)


def _tc_matmul_body(x_ref, wt_ref, kw_ref, b_ref, o_ref):
    acc = lax.dot_general(
        x_ref[:], wt_ref[:], (((1,), (1,)), ((), ())),
        preferred_element_type=jnp.float32)
    o_ref[:] = acc * kw_ref[:] + b_ref[:]


def _tc_matmul(x, wt, kw, b2):
    return pl.pallas_call(
        _tc_matmul_body,
        out_shape=jax.ShapeDtypeStruct((B, U), jnp.float32),
    )(x, wt, kw, b2)


def kernel(x, dendrites, kernel, dendriticW, bias):
    duds = jnp.transpose(dendrites, (2, 0, 1)).reshape(U * D * S)
    wt = _sc_scatter(duds, dendriticW)
    return _tc_matmul(x, wt, kernel, bias.reshape(1, U))


# speedup vs baseline: 1.2665x; 1.0277x over previous
import functools

import jax
import jax.numpy as jnp
from jax import lax
from jax.experimental import pallas as pl
from jax.experimental.pallas import tpu as pltpu
from jax.experimental.pallas import tpu_sc as plsc

B, C, U, D, S = 1024, 256, 128, 16, 16
NC, NS = 2, 16
NW = NC * NS
UPW = U // NW
L = 16


def _sc_scatter_body(duds_hbm, dw_hbm, wt_hbm, idx_v, dw_v, w_v, sem1, sem2):
    wid = lax.axis_index("s") * NC + lax.axis_index("c")
    base = wid * UPW
    cp1 = pltpu.async_copy(
        duds_hbm.at[pl.ds(base * D * S, UPW * D * S)], idx_v, sem1)
    cp2 = pltpu.async_copy(dw_hbm, dw_v, sem2)
    cp1.wait()
    cp2.wait()
    lane = lax.iota(jnp.int32, L)

    def unit_body(j, _):
        rows = lane * 0 + j
        w16 = plsc.load_gather(dw_v, [lane, lane * 0 + (base + j)])

        def d_body(d, _):
            idx = idx_v[pl.ds(j * D * S + d * S, L)]
            plsc.store_scatter(w_v, [rows, idx], w16)
            return 0

        return lax.fori_loop(0, D, d_body, 0)

    lax.fori_loop(0, UPW, unit_body, 0)
    pltpu.sync_copy(w_v, wt_hbm.at[pl.ds(base, UPW)])


@functools.partial(
    pl.kernel,
    mesh=plsc.VectorSubcoreMesh(core_axis_name="c", subcore_axis_name="s"),
    out_type=jax.ShapeDtypeStruct((U, C), jnp.float32),
    scratch_types=[
        pltpu.VMEM((UPW * D * S,), jnp.int32),
        pltpu.VMEM((S, U), jnp.float32),
        pltpu.VMEM((UPW, C), jnp.float32),
        pltpu.SemaphoreType.DMA,
        pltpu.SemaphoreType.DMA,
    ],
    compiler_params=pltpu.CompilerParams(needs_layout_passes=False),
)
def _sc_scatter(duds_hbm, dw_hbm, wt_hbm, idx_v, dw_v, w_v, sem1, sem2):
    _sc_scatter_body(duds_hbm, dw_hbm, wt_hbm, idx_v, dw_v, w_v, sem1, sem2)


def _tc_matmul_body(x_ref, wt_ref, kw_ref, b_ref, o_ref):
    acc = lax.dot_general(
        x_ref[:], wt_ref[:], (((1,), (1,)), ((), ())),
        preferred_element_type=jnp.float32)
    o_ref[:] = acc * kw_ref[:] + b_ref[:]


def _tc_matmul(x, wt, kw, b2):
    return pl.pallas_call(
        _tc_matmul_body,
        out_shape=jax.ShapeDtypeStruct((B, U), jnp.float32),
    )(x, wt, kw, b2)


def kernel(x, dendrites, kernel, dendriticW, bias):
    duds = jnp.transpose(dendrites, (2, 0, 1)).reshape(U * D * S)
    wt = _sc_scatter(duds, dendriticW)
    return _tc_matmul(x, wt, kernel, bias.reshape(1, U))


# speedup vs baseline: 1.3418x; 1.0595x over previous
import functools

import jax
import jax.numpy as jnp
from jax import lax
from jax.experimental import pallas as pl
from jax.experimental.pallas import tpu as pltpu
from jax.experimental.pallas import tpu_sc as plsc

B, C, U, D, S = 1024, 256, 128, 16, 16
NC, NS = 1, 16
NW = NC * NS
UPW = U // NW
L = 16


def _sc_scatter_body(duds_hbm, dw_hbm, wt_hbm, idx_v, dw_v, w_v, sem1, sem2):
    wid = lax.axis_index("s")
    base = wid * UPW
    cp1 = pltpu.async_copy(
        duds_hbm.at[pl.ds(base * D * S, UPW * D * S)], idx_v, sem1)
    cp2 = pltpu.async_copy(dw_hbm, dw_v, sem2)
    cp1.wait()
    cp2.wait()
    lane = lax.iota(jnp.int32, L)

    def unit_body(j, _):
        rows = lane * 0 + j
        w16 = plsc.load_gather(dw_v, [lane, lane * 0 + (base + j)])

        def d_body(d, _):
            idx = idx_v[pl.ds(j * D * S + d * S, L)]
            plsc.store_scatter(w_v, [rows, idx], w16)
            return 0

        return lax.fori_loop(0, D, d_body, 0)

    lax.fori_loop(0, UPW, unit_body, 0)
    pltpu.sync_copy(w_v, wt_hbm.at[pl.ds(base, UPW)])


@functools.partial(
    pl.kernel,
    mesh=plsc.VectorSubcoreMesh(
        core_axis_name="c", subcore_axis_name="s", num_cores=NC),
    out_type=jax.ShapeDtypeStruct((U, C), jnp.float32),
    scratch_types=[
        pltpu.VMEM((UPW * D * S,), jnp.int32),
        pltpu.VMEM((S, U), jnp.float32),
        pltpu.VMEM((UPW, C), jnp.float32),
        pltpu.SemaphoreType.DMA,
        pltpu.SemaphoreType.DMA,
    ],
    compiler_params=pltpu.CompilerParams(needs_layout_passes=False),
)
def _sc_scatter(duds_hbm, dw_hbm, wt_hbm, idx_v, dw_v, w_v, sem1, sem2):
    _sc_scatter_body(duds_hbm, dw_hbm, wt_hbm, idx_v, dw_v, w_v, sem1, sem2)


def _tc_matmul_body(x_ref, wt_ref, kw_ref, b_ref, o_ref):
    acc = lax.dot_general(
        x_ref[:], wt_ref[:], (((1,), (1,)), ((), ())),
        preferred_element_type=jnp.float32)
    o_ref[:] = acc * kw_ref[:] + b_ref[:]


def _tc_matmul(x, wt, kw, b2):
    return pl.pallas_call(
        _tc_matmul_body,
        out_shape=jax.ShapeDtypeStruct((B, U), jnp.float32),
    )(x, wt, kw, b2)


def kernel(x, dendrites, kernel, dendriticW, bias):
    duds = jnp.transpose(dendrites, (2, 0, 1)).reshape(U * D * S)
    wt = _sc_scatter(duds, dendriticW)
    return _tc_matmul(x, wt, kernel, bias.reshape(1, U))
